# baseline (speedup 1.0000x reference)
import jax
import jax.numpy as jnp
from jax.experimental import pallas as pl
from jax.experimental.pallas import tpu as pltpu

_R = 64
_K = 8


def _topk_mask(a):
    idx = jax.lax.broadcasted_iota(jnp.int32, a.shape, 1)
    mask = jnp.zeros(a.shape, dtype=jnp.bool_)
    work = a
    for _ in range(_K):
        am = jnp.argmax(work, axis=1)
        sel = idx == am[:, None]
        mask = jnp.logical_or(mask, sel)
        work = jnp.where(sel, -jnp.inf, work)
    return mask


def _body(x_ref, a_ref, b_ref, d_ref, out_ref):
    x = x_ref[...].astype(jnp.bfloat16)
    dn = (((1,), (1,)), ((), ()))
    y = jax.lax.dot_general(
        x, a_ref[...], dn, preferred_element_type=jnp.float32)
    bt = x.shape[0]
    h = bt // 2
    for s in range(2):
        ys = y[s * h:(s + 1) * h, :]
        a = jnp.abs(ys + d_ref[...])
        mask = _topk_mask(a)
        act = jnp.where(mask, ys + ys, 0.0).astype(jnp.bfloat16)
        out_ref[s * h:(s + 1) * h, :] = jax.lax.dot_general(
            act, b_ref[...], dn, preferred_element_type=jnp.float32)


@jax.jit
def kernel(x, A, B, d):
    n, in_f = x.shape
    out_f = B.shape[0]
    bt = 512
    grid = (n // bt,)
    return pl.pallas_call(
        _body,
        grid=grid,
        in_specs=[
            pl.BlockSpec((bt, in_f), lambda i: (i, 0)),
            pl.BlockSpec((_R, in_f), lambda i: (0, 0)),
            pl.BlockSpec((out_f, _R), lambda i: (0, 0)),
            pl.BlockSpec((1, _R), lambda i: (0, 0)),
        ],
        out_specs=pl.BlockSpec((bt, out_f), lambda i: (i, 0)),
        out_shape=jax.ShapeDtypeStruct((n, out_f), jnp.float32),
        compiler_params=pltpu.CompilerParams(
            dimension_semantics=("parallel",)),
    )(x, A.astype(jnp.bfloat16), B.astype(jnp.bfloat16), d.reshape(1, _R))


# speedup vs baseline: 1.0129x; 1.0129x over previous
import jax
import jax.numpy as jnp
from jax.experimental import pallas as pl
from jax.experimental.pallas import tpu as pltpu

_R = 64
_K = 8


def _body(x_ref, a_ref, b_ref, d_ref, out_ref):
    dn = (((1,), (1,)), ((), ()))
    y = jax.lax.dot_general(
        x_ref[...], a_ref[...], dn,
        preferred_element_type=jnp.float32)
    work = jnp.abs(y + d_ref[...])
    idx = jax.lax.broadcasted_iota(jnp.int32, work.shape, 1)
    for _ in range(_K):
        am = jnp.argmax(work, axis=1)
        work = jnp.where(idx == am[:, None], -jnp.inf, work)
    act = jnp.where(jnp.isneginf(work), y + y, 0.0).astype(jnp.bfloat16)
    out_ref[...] = jax.lax.dot_general(
        act, b_ref[...], dn, preferred_element_type=jnp.float32)


@jax.jit
def kernel(x, A, B, d):
    n, in_f = x.shape
    out_f = B.shape[0]
    bt = 512
    grid = (n // bt,)
    return pl.pallas_call(
        _body,
        grid=grid,
        in_specs=[
            pl.BlockSpec((bt, in_f), lambda i: (i, 0)),
            pl.BlockSpec((_R, in_f), lambda i: (0, 0)),
            pl.BlockSpec((out_f, _R), lambda i: (0, 0)),
            pl.BlockSpec((1, _R), lambda i: (0, 0)),
        ],
        out_specs=pl.BlockSpec((bt, out_f), lambda i: (i, 0)),
        out_shape=jax.ShapeDtypeStruct((n, out_f), jnp.float32),
        compiler_params=pltpu.CompilerParams(
            dimension_semantics=("parallel",)),
    )(x, A, B.astype(jnp.bfloat16), d.reshape(1, _R))
